# baseline (speedup 1.0000x reference)
import functools

import jax
import jax.numpy as jnp
from jax.experimental import pallas as pl
from jax.experimental.pallas import tpu as pltpu


def _conv_stats_pool_kernel(x_ref, w_ref, b_ref, maxp_ref, minp_ref, ps_ref):
    _, _, _, Cin = x_ref.shape
    _, Hh, Wh, Cout = maxp_ref.shape
    H, W = 2 * Hh, 2 * Wh

    cols = []
    for kh in range(3):
        for kw in range(3):
            cols.append(x_ref[0, kh:kh + H, kw:kw + W, :].reshape(H * W, Cin))
    patches = jnp.concatenate(cols, axis=-1)

    acc = jnp.dot(patches, w_ref[...], preferred_element_type=jnp.float32)
    acc = jnp.maximum(acc + b_ref[...], 0.0)

    s = jnp.sum(acc, axis=0, keepdims=True)
    sq = jnp.sum(acc * acc, axis=0, keepdims=True)
    ps_ref[...] = jnp.concatenate([s, sq], axis=0).reshape(1, 2, Cout)

    r = acc.reshape(Hh, 2, Wh, 2, Cout)
    maxp_ref[...] = jnp.max(jnp.max(r, axis=3), axis=1).reshape(1, Hh, Wh, Cout)
    minp_ref[...] = jnp.min(jnp.min(r, axis=3), axis=1).reshape(1, Hh, Wh, Cout)


def _bn_select_kernel(maxp_ref, minp_ref, stats_ref, g_ref, beta_ref, o_ref, *, inv_count):
    _, Hh, Wh, Cout = o_ref.shape
    mean = stats_ref[0:1, :] * inv_count
    var = stats_ref[1:2, :] * inv_count - mean * mean
    scale = g_ref[...] * jax.lax.rsqrt(var + 1e-5)
    shift = beta_ref[...] - mean * scale
    scale4 = scale.reshape(1, 1, 1, Cout)
    sel = jnp.where(scale4 >= 0.0, maxp_ref[...], minp_ref[...])
    o_ref[...] = sel * scale4 + shift.reshape(1, 1, 1, Cout)


def kernel(x_nchw, w_oihw, bias, gamma, beta):
    N, Cin, H, W = x_nchw.shape
    Cout = w_oihw.shape[0]
    Hh, Wh = H // 2, W // 2

    x_nhwc = jnp.transpose(x_nchw, (0, 2, 3, 1))
    x_pad = jnp.pad(x_nhwc, ((0, 0), (1, 1), (1, 1), (0, 0))).astype(jnp.bfloat16)
    w_flat = (
        jnp.transpose(w_oihw, (2, 3, 1, 0)).reshape(9 * Cin, Cout).astype(jnp.bfloat16)
    )
    b2 = bias.reshape(1, Cout).astype(jnp.float32)
    g2 = gamma.reshape(1, Cout).astype(jnp.float32)
    be2 = beta.reshape(1, Cout).astype(jnp.float32)

    maxp, minp, pstats = pl.pallas_call(
        _conv_stats_pool_kernel,
        grid=(N,),
        in_specs=[
            pl.BlockSpec((1, H + 2, W + 2, Cin), lambda n: (n, 0, 0, 0)),
            pl.BlockSpec((9 * Cin, Cout), lambda n: (0, 0)),
            pl.BlockSpec((1, Cout), lambda n: (0, 0)),
        ],
        out_specs=(
            pl.BlockSpec((1, Hh, Wh, Cout), lambda n: (n, 0, 0, 0)),
            pl.BlockSpec((1, Hh, Wh, Cout), lambda n: (n, 0, 0, 0)),
            pl.BlockSpec((1, 2, Cout), lambda n: (n, 0, 0)),
        ),
        out_shape=(
            jax.ShapeDtypeStruct((N, Hh, Wh, Cout), jnp.float32),
            jax.ShapeDtypeStruct((N, Hh, Wh, Cout), jnp.float32),
            jax.ShapeDtypeStruct((N, 2, Cout), jnp.float32),
        ),
        compiler_params=pltpu.CompilerParams(dimension_semantics=("parallel",)),
    )(x_pad, w_flat, b2)

    stats = jnp.sum(pstats, axis=0)
    inv_count = 1.0 / float(N * H * W)

    out_nhwc = pl.pallas_call(
        functools.partial(_bn_select_kernel, inv_count=inv_count),
        grid=(N,),
        in_specs=[
            pl.BlockSpec((1, Hh, Wh, Cout), lambda n: (n, 0, 0, 0)),
            pl.BlockSpec((1, Hh, Wh, Cout), lambda n: (n, 0, 0, 0)),
            pl.BlockSpec((2, Cout), lambda n: (0, 0)),
            pl.BlockSpec((1, Cout), lambda n: (0, 0)),
            pl.BlockSpec((1, Cout), lambda n: (0, 0)),
        ],
        out_specs=pl.BlockSpec((1, Hh, Wh, Cout), lambda n: (n, 0, 0, 0)),
        out_shape=jax.ShapeDtypeStruct((N, Hh, Wh, Cout), jnp.float32),
        compiler_params=pltpu.CompilerParams(dimension_semantics=("parallel",)),
    )(maxp, minp, stats, g2, be2)

    return jnp.transpose(out_nhwc, (0, 3, 1, 2))


# speedup vs baseline: 2.0282x; 2.0282x over previous
import functools

import jax
import jax.numpy as jnp
from jax.experimental import pallas as pl
from jax.experimental.pallas import tpu as pltpu


def _conv_stats_pool_kernel(x_ref, w_ref, b_ref, sgn_ref, pooled_ref, ps_ref, scr_ref):
    _, _, _, Cin = x_ref.shape
    _, Hh, Wh, Cout = pooled_ref.shape
    H, W = 2 * Hh, 2 * Wh

    cols = []
    for kh in range(3):
        for kw in range(3):
            cols.append(x_ref[0, kh:kh + H, kw:kw + W, :].reshape(H * W, Cin))
    patches = jnp.concatenate(cols, axis=-1)

    acc = jnp.dot(patches, w_ref[...], preferred_element_type=jnp.float32)
    acc = jnp.maximum(acc + b_ref[...], 0.0)

    s = jnp.sum(acc, axis=0, keepdims=True)
    sq = jnp.sum(acc * acc, axis=0, keepdims=True)
    ps_ref[...] = jnp.concatenate([s, sq], axis=0).reshape(1, 2, Cout)

    scr_ref[...] = acc.reshape(H, W, Cout)
    even = scr_ref[:, pl.ds(0, Wh, stride=2), :]
    odd = scr_ref[:, pl.ds(1, Wh, stride=2), :]
    mx = jnp.maximum(even, odd).reshape(Hh, 2, Wh, Cout)
    mn = jnp.minimum(even, odd).reshape(Hh, 2, Wh, Cout)
    mx2 = jnp.max(mx, axis=1)
    mn2 = jnp.min(mn, axis=1)
    sel = jnp.where(sgn_ref[...].reshape(1, 1, Cout) >= 0.0, mx2, mn2)
    pooled_ref[...] = sel.reshape(1, Hh, Wh, Cout)


def _bn_apply_kernel(pooled_ref, stats_ref, g_ref, beta_ref, o_ref, *, inv_count):
    _, Hh, Wh, Cout = o_ref.shape
    mean = stats_ref[0:1, :] * inv_count
    var = stats_ref[1:2, :] * inv_count - mean * mean
    scale = g_ref[...] * jax.lax.rsqrt(var + 1e-5)
    shift = beta_ref[...] - mean * scale
    o_ref[...] = pooled_ref[...] * scale.reshape(1, 1, 1, Cout) + shift.reshape(
        1, 1, 1, Cout
    )


def kernel(x_nchw, w_oihw, bias, gamma, beta):
    N, Cin, H, W = x_nchw.shape
    Cout = w_oihw.shape[0]
    Hh, Wh = H // 2, W // 2

    x_nhwc = jnp.transpose(x_nchw, (0, 2, 3, 1)).astype(jnp.float32)
    x_pad = jnp.pad(x_nhwc, ((0, 0), (1, 1), (1, 1), (0, 0)))
    w_flat = jnp.transpose(w_oihw, (2, 3, 1, 0)).reshape(9 * Cin, Cout).astype(jnp.float32)
    b2 = bias.reshape(1, Cout).astype(jnp.float32)
    g2 = gamma.reshape(1, Cout).astype(jnp.float32)
    be2 = beta.reshape(1, Cout).astype(jnp.float32)

    pooled, pstats = pl.pallas_call(
        _conv_stats_pool_kernel,
        grid=(N,),
        in_specs=[
            pl.BlockSpec((1, H + 2, W + 2, Cin), lambda n: (n, 0, 0, 0)),
            pl.BlockSpec((9 * Cin, Cout), lambda n: (0, 0)),
            pl.BlockSpec((1, Cout), lambda n: (0, 0)),
            pl.BlockSpec((1, Cout), lambda n: (0, 0)),
        ],
        out_specs=(
            pl.BlockSpec((1, Hh, Wh, Cout), lambda n: (n, 0, 0, 0)),
            pl.BlockSpec((1, 2, Cout), lambda n: (n, 0, 0)),
        ),
        out_shape=(
            jax.ShapeDtypeStruct((N, Hh, Wh, Cout), jnp.float32),
            jax.ShapeDtypeStruct((N, 2, Cout), jnp.float32),
        ),
        scratch_shapes=[pltpu.VMEM((H, W, Cout), jnp.float32)],
        compiler_params=pltpu.CompilerParams(dimension_semantics=("parallel",)),
    )(x_pad, w_flat, b2, g2)

    stats = jnp.sum(pstats, axis=0)
    inv_count = 1.0 / float(N * H * W)

    out_nhwc = pl.pallas_call(
        functools.partial(_bn_apply_kernel, inv_count=inv_count),
        grid=(N,),
        in_specs=[
            pl.BlockSpec((1, Hh, Wh, Cout), lambda n: (n, 0, 0, 0)),
            pl.BlockSpec((2, Cout), lambda n: (0, 0)),
            pl.BlockSpec((1, Cout), lambda n: (0, 0)),
            pl.BlockSpec((1, Cout), lambda n: (0, 0)),
        ],
        out_specs=pl.BlockSpec((1, Hh, Wh, Cout), lambda n: (n, 0, 0, 0)),
        out_shape=jax.ShapeDtypeStruct((N, Hh, Wh, Cout), jnp.float32),
        compiler_params=pltpu.CompilerParams(dimension_semantics=("parallel",)),
    )(pooled, stats, g2, be2)

    return jnp.transpose(out_nhwc, (0, 3, 1, 2))


# speedup vs baseline: 2.0705x; 1.0209x over previous
import functools

import jax
import jax.numpy as jnp
from jax.experimental import pallas as pl
from jax.experimental.pallas import tpu as pltpu


def _conv_stats_pool_kernel(x_ref, w_ref, b_ref, sgn_ref, pooled_ref, ps_ref, scr_ref):
    _, _, _, Cin = x_ref.shape
    _, Hh, Wh, Cout = pooled_ref.shape
    H, W = 2 * Hh, 2 * Wh

    cols = []
    for kh in range(3):
        for kw in range(3):
            cols.append(x_ref[0, kh:kh + H, kw:kw + W, :].reshape(H * W, Cin))
    patches = jnp.concatenate(cols, axis=-1)

    acc = jnp.dot(patches, w_ref[...], preferred_element_type=jnp.float32)
    acc = jnp.maximum(acc + b_ref[...], 0.0)

    s = jnp.sum(acc, axis=0, keepdims=True)
    sq = jnp.sum(acc * acc, axis=0, keepdims=True)
    ps_ref[...] = jnp.concatenate([s, sq], axis=0).reshape(1, 2, Cout)

    scr_ref[...] = acc.reshape(H, W, Cout)
    even = scr_ref[:, pl.ds(0, Wh, stride=2), :]
    odd = scr_ref[:, pl.ds(1, Wh, stride=2), :]
    mx = jnp.maximum(even, odd).reshape(Hh, 2, Wh, Cout)
    mn = jnp.minimum(even, odd).reshape(Hh, 2, Wh, Cout)
    mx2 = jnp.max(mx, axis=1)
    mn2 = jnp.min(mn, axis=1)
    sel = jnp.where(sgn_ref[...].reshape(1, 1, Cout) >= 0.0, mx2, mn2)
    pooled_ref[...] = sel.reshape(1, Hh, Wh, Cout)


def _bn_apply_kernel(pooled_ref, stats_ref, g_ref, beta_ref, o_ref, *, inv_count):
    _, Cout, HW = o_ref.shape
    _, Hh, Wh, _ = pooled_ref.shape
    mean = stats_ref[0:1, :] * inv_count
    var = stats_ref[1:2, :] * inv_count - mean * mean
    scale = g_ref[...] * jax.lax.rsqrt(var + 1e-5)
    shift = beta_ref[...] - mean * scale
    val = pooled_ref[...].reshape(HW, Cout) * scale + shift
    o_ref[...] = jnp.transpose(val, (1, 0)).reshape(1, Cout, HW)


def kernel(x_nchw, w_oihw, bias, gamma, beta):
    N, Cin, H, W = x_nchw.shape
    Cout = w_oihw.shape[0]
    Hh, Wh = H // 2, W // 2

    x_nhwc = jnp.transpose(x_nchw, (0, 2, 3, 1)).astype(jnp.float32)
    x_pad = jnp.pad(x_nhwc, ((0, 0), (1, 1), (1, 1), (0, 0)))
    w_flat = jnp.transpose(w_oihw, (2, 3, 1, 0)).reshape(9 * Cin, Cout).astype(jnp.float32)
    b2 = bias.reshape(1, Cout).astype(jnp.float32)
    g2 = gamma.reshape(1, Cout).astype(jnp.float32)
    be2 = beta.reshape(1, Cout).astype(jnp.float32)

    pooled, pstats = pl.pallas_call(
        _conv_stats_pool_kernel,
        grid=(N,),
        in_specs=[
            pl.BlockSpec((1, H + 2, W + 2, Cin), lambda n: (n, 0, 0, 0)),
            pl.BlockSpec((9 * Cin, Cout), lambda n: (0, 0)),
            pl.BlockSpec((1, Cout), lambda n: (0, 0)),
            pl.BlockSpec((1, Cout), lambda n: (0, 0)),
        ],
        out_specs=(
            pl.BlockSpec((1, Hh, Wh, Cout), lambda n: (n, 0, 0, 0)),
            pl.BlockSpec((1, 2, Cout), lambda n: (n, 0, 0)),
        ),
        out_shape=(
            jax.ShapeDtypeStruct((N, Hh, Wh, Cout), jnp.float32),
            jax.ShapeDtypeStruct((N, 2, Cout), jnp.float32),
        ),
        scratch_shapes=[pltpu.VMEM((H, W, Cout), jnp.float32)],
        compiler_params=pltpu.CompilerParams(dimension_semantics=("parallel",)),
    )(x_pad, w_flat, b2, g2)

    stats = jnp.sum(pstats, axis=0)
    inv_count = 1.0 / float(N * H * W)

    out_nchw = pl.pallas_call(
        functools.partial(_bn_apply_kernel, inv_count=inv_count),
        grid=(N,),
        in_specs=[
            pl.BlockSpec((1, Hh, Wh, Cout), lambda n: (n, 0, 0, 0)),
            pl.BlockSpec((2, Cout), lambda n: (0, 0)),
            pl.BlockSpec((1, Cout), lambda n: (0, 0)),
            pl.BlockSpec((1, Cout), lambda n: (0, 0)),
        ],
        out_specs=pl.BlockSpec((1, Cout, Hh * Wh), lambda n: (n, 0, 0)),
        out_shape=jax.ShapeDtypeStruct((N, Cout, Hh * Wh), jnp.float32),
        compiler_params=pltpu.CompilerParams(dimension_semantics=("parallel",)),
    )(pooled, stats, g2, be2)

    return out_nchw.reshape(N, Cout, Hh, Wh)
